# Initial kernel scaffold; baseline (speedup 1.0000x reference)
import functools

import jax
import jax.numpy as jnp
import numpy as np
from jax import lax
from jax.experimental import pallas as pl
from jax.experimental.pallas import tpu as pltpu
from jax.experimental.pallas import tpu_sc as plsc

RATIO = 2

_NC = 2
_NS = 16
_NW = _NC * _NS

_NUMS = 262144
_INDEX_NUM = _NUMS // RATIO
_IDX = np.asarray(
    jax.random.permutation(jax.random.key(42), _NUMS)[:_INDEX_NUM]
).astype(np.int32)

_CH = 1024
_B_PER_W = _INDEX_NUM // _NW
_N_CHUNKS = _B_PER_W // _CH


def _make_gather(V, D, B):
    mesh = plsc.VectorSubcoreMesh(core_axis_name="c", subcore_axis_name="s")

    @functools.partial(
        pl.kernel,
        mesh=mesh,
        out_type=jax.ShapeDtypeStruct((B, D), jnp.float32),
        scratch_types=[
            pltpu.VMEM((_CH,), jnp.int32),
            pltpu.VMEM((_CH, D), jnp.float32),
            pltpu.SemaphoreType.DMA,
        ],
    )
    def gather_kernel(table_hbm, idx_hbm, out_hbm, idx_v, rows_v, sem):
        wid = lax.axis_index("s") * _NC + lax.axis_index("c")
        base = wid * _B_PER_W
        for i in range(_N_CHUNKS):
            off = base + i * _CH
            pltpu.sync_copy(idx_hbm.at[pl.ds(off, _CH)], idx_v)
            pltpu.async_copy(table_hbm.at[idx_v], rows_v, sem).wait()
            pltpu.sync_copy(rows_v, out_hbm.at[pl.ds(off, _CH)])

    return gather_kernel


_gather = _make_gather(_NUMS, 96, _INDEX_NUM)


def kernel(x):
    b, c, nums = x.shape
    xT = x[0].T
    idx = jnp.asarray(_IDX)
    out_t = _gather(xT, idx)
    return out_t.T[None]


# speedup vs baseline: 1.6691x; 1.6691x over previous
import functools

import jax
import jax.numpy as jnp
import numpy as np
from jax import lax
from jax.experimental import pallas as pl
from jax.experimental.pallas import tpu as pltpu
from jax.experimental.pallas import tpu_sc as plsc

RATIO = 2

_NC = 2
_NS = 16
_NW = _NC * _NS

_NUMS = 262144
_INDEX_NUM = _NUMS // RATIO
_IDX_CACHE = {}


def _sampled_idx(nums, index_num):
    if nums not in _IDX_CACHE:
        def _compute():
            perm = jax.random.permutation(jax.random.key(42), nums)
            return perm[:index_num].astype(jnp.int32)

        cpu = jax.local_devices(backend="cpu")[0]
        with jax.ensure_compile_time_eval(), jax.default_device(cpu):
            _IDX_CACHE[nums] = np.asarray(jax.jit(_compute)())
    return _IDX_CACHE[nums]

_CH = 512
_B_PER_W = _INDEX_NUM // _NW
_N_CHUNKS = _B_PER_W // _CH


@functools.lru_cache(maxsize=None)
def _make_gather(V, D, B):
    mesh = plsc.VectorSubcoreMesh(core_axis_name="c", subcore_axis_name="s")

    @functools.partial(
        pl.kernel,
        mesh=mesh,
        out_type=jax.ShapeDtypeStruct((B, D), jnp.float32),
        scratch_types=[
            pltpu.VMEM((_CH,), jnp.int32),
            pltpu.VMEM((_CH, D), jnp.float32),
            pltpu.SemaphoreType.DMA,
        ],
    )
    def gather_kernel(table_hbm, idx_hbm, out_hbm, idx_v, rows_v, sem):
        wid = lax.axis_index("s") * _NC + lax.axis_index("c")
        base = wid * _B_PER_W
        for i in range(_N_CHUNKS):
            off = base + i * _CH
            pltpu.sync_copy(idx_hbm.at[pl.ds(off, _CH)], idx_v)
            pltpu.async_copy(table_hbm.at[idx_v], rows_v, sem).wait()
            pltpu.sync_copy(rows_v, out_hbm.at[pl.ds(off, _CH)])

    return gather_kernel


def kernel(x):
    b, c, nums = x.shape
    cp = 128
    xT = jnp.pad(x[0].T, ((0, 0), (0, cp - c)))
    idx = jnp.asarray(_sampled_idx(nums, nums // RATIO))
    out_t = _make_gather(nums, cp, nums // RATIO)(xT, idx)
    return out_t[:, :c].T[None]
